# Initial kernel scaffold; baseline (speedup 1.0000x reference)
import jax
import jax.numpy as jnp
from jax.experimental import pallas as pl


def kernel(tile_type, normalized_steps, param_list, sensor_mask, normalized_unit_counts, normalized_unit_counts_opp, normalized_unit_energys_max_grid, normalized_unit_energys_max_grid_opp, grid_probability_of_being_an_energy_point_based_on_no_reward, grid_max_probability_of_being_an_energy_point_based_on_positive_rewards, grid_avg_probability_of_being_an_energy_point_based_on_positive_rewards, grid_probability_of_being_energy_point_based_on_relic_positions, value_of_sapping_grid, embed_table):
    raise NotImplementedError("write your pallas kernel here")


# speedup vs baseline: 73.0004x; 73.0004x over previous
import jax
import jax.numpy as jnp
from jax.experimental import pallas as pl
from jax.experimental.pallas import tpu as pltpu

B, H, W, P = 4096, 24, 24, 10
HW = H * W
EC = 5
C = EC + 1 + P + 10
BB = 1024
NBC = B // BB


def _tc_body(tt_ref, st_ref, par_ref, g0, g1, g2, g3, g4, g5, g6, g7, g8, g9,
             tbl_ref, out_ref):
    g_refs = (g0, g1, g2, g3, g4, g5, g6, g7, g8, g9)
    tt = tt_ref[...]
    for c in range(EC):
        t0 = tbl_ref[0, c]
        t1 = tbl_ref[1, c]
        t2 = tbl_ref[2, c]
        t3 = tbl_ref[3, c]
        v = jnp.where(tt < 2, jnp.where(tt == 0, t0, t1),
                      jnp.where(tt == 2, t2, t3))
        out_ref[0, c] = v
    out_ref[0, EC] = jnp.broadcast_to(st_ref[...], (W, BB))
    for k in range(P):
        out_ref[0, EC + 1 + k] = jnp.broadcast_to(par_ref[k:k + 1, :], (W, BB))
    for g in range(10):
        out_ref[0, EC + 1 + P + g] = g_refs[g][...]


@jax.jit
def _encode(tt, st, par, grids, tbl):
    hw_spec = pl.BlockSpec((W, BB), lambda h, j: (h, j))
    run = pl.pallas_call(
        _tc_body,
        grid=(H, NBC),
        in_specs=[
            hw_spec,
            pl.BlockSpec((1, BB), lambda h, j: (0, j)),
            pl.BlockSpec((P, BB), lambda h, j: (0, j)),
        ] + [hw_spec] * 10 + [
            pl.BlockSpec(memory_space=pltpu.SMEM),
        ],
        out_specs=pl.BlockSpec((1, C, W, BB), lambda h, j: (h, 0, 0, j)),
        out_shape=jax.ShapeDtypeStruct((H, C, W, B), jnp.float32),
    )
    return run(tt, st, par, *grids, tbl)


def kernel(tile_type, normalized_steps, param_list,
           sensor_mask, normalized_unit_counts, normalized_unit_counts_opp,
           normalized_unit_energys_max_grid, normalized_unit_energys_max_grid_opp,
           grid_probability_of_being_an_energy_point_based_on_no_reward,
           grid_max_probability_of_being_an_energy_point_based_on_positive_rewards,
           grid_avg_probability_of_being_an_energy_point_based_on_positive_rewards,
           grid_probability_of_being_energy_point_based_on_relic_positions,
           value_of_sapping_grid, embed_table):
    grids = (sensor_mask, normalized_unit_counts, normalized_unit_counts_opp,
             normalized_unit_energys_max_grid, normalized_unit_energys_max_grid_opp,
             grid_probability_of_being_an_energy_point_based_on_no_reward,
             grid_max_probability_of_being_an_energy_point_based_on_positive_rewards,
             grid_avg_probability_of_being_an_energy_point_based_on_positive_rewards,
             grid_probability_of_being_energy_point_based_on_relic_positions,
             value_of_sapping_grid)
    tt = jnp.transpose(tile_type.astype(jnp.int32), (1, 2, 0)).reshape(HW, B)
    st = normalized_steps.astype(jnp.float32).reshape(1, B)
    par = jnp.transpose(param_list, (1, 0))
    gr = tuple(jnp.transpose(g, (1, 2, 0)).reshape(HW, B) for g in grids)
    out = _encode(tt, st, par, gr, embed_table)
    return jnp.transpose(out, (3, 0, 2, 1))
